# baseline (speedup 1.0000x reference)
import functools

import jax
import jax.numpy as jnp
from jax import lax
from jax.experimental import pallas as pl
from jax.experimental.pallas import tpu as pltpu
from jax.experimental.pallas import tpu_sc as plsc

N = 10000
E = 320000
D = 128
G = 64
GSZ = 5

NC = 2
NS = 16
NW = NC * NS
CHUNK = 128
CPW = 80
NCHUNKS_P = NW * CPW
EP = NCHUNKS_P * CHUNK
NSINK = 8
RB = 200
NB = N // RB


def _sc_agg_body(table_hbm, src_hbm, dst_hbm, out_hbm,
                 src_v, dst_v, rows0_v, rows1_v, acc_sh,
                 gsem0, gsem1):
    c = lax.axis_index("c")
    s = lax.axis_index("s")
    w = s * NC + c

    zero16 = jnp.zeros((16,), jnp.float32)

    def _zero_body(i, carry):
        for j in range(D // 16):
            rows0_v[i, pl.ds(j * 16, 16)] = zero16
        return carry

    lax.fori_loop(0, CHUNK, _zero_body, 0)

    nzb = N // CHUNK

    def _zinit(k, carry):
        bid = s + NS * k

        @pl.when(bid < nzb)
        def _():
            off = pl.multiple_of(bid * CHUNK, 8)
            pltpu.sync_copy(rows0_v, acc_sh.at[pl.ds(off, CHUNK)])

        return carry

    lax.fori_loop(0, (nzb + NS - 1) // NS, _zinit, 0)

    @pl.when(s == 0)
    def _():
        pltpu.sync_copy(rows0_v.at[pl.ds(0, 16)],
                        acc_sh.at[pl.ds(nzb * CHUNK, 16)])

    plsc.subcore_barrier()

    def _gather(k, buf, sem):
        return pltpu.async_copy(table_hbm.at[src_v.at[k]], buf, sem)

    def _gwait(k, buf, sem):
        pltpu.make_async_copy(table_hbm.at[src_v.at[k]], buf, sem).wait()

    def _scat(k, buf):
        pltpu.sync_copy(buf, acc_sh.at[dst_v.at[k]], add=True)

    half = CPW // 2
    for p in range(2):
        iw = pl.multiple_of(w * CPW + p * half, 8)
        pltpu.sync_copy(src_hbm.at[pl.ds(iw, half)], src_v)
        pltpu.sync_copy(dst_hbm.at[pl.ds(iw, half)], dst_v)
        def _edge_body(i, carry):
            _gather(i, rows0_v, gsem0).wait()
            _scat(i, rows0_v)
            return carry

        lax.fori_loop(0, half, _edge_body, 0)

    plsc.subcore_barrier()

    def _dump(k, carry):
        bid = s + NS * k

        @pl.when(bid < NB)
        def _():
            off = pl.multiple_of(bid * RB, 8)
            pltpu.sync_copy(acc_sh.at[pl.ds(off, RB)],
                            out_hbm.at[c, pl.ds(off, RB)])

        return carry

    lax.fori_loop(0, (NB + NS - 1) // NS, _dump, 0)


def _sc_pass(table, src, dst):
    mesh = plsc.VectorSubcoreMesh(core_axis_name="c", subcore_axis_name="s")
    kern = pl.kernel(
        _sc_agg_body,
        mesh=mesh,
        out_type=jax.ShapeDtypeStruct((NC, N, D), jnp.float32),
        scratch_types=[
            pltpu.VMEM((CPW // 2, CHUNK), jnp.int32),
            pltpu.VMEM((CPW // 2, CHUNK), jnp.int32),
            pltpu.VMEM((CHUNK, D), jnp.float32),
            pltpu.VMEM((CHUNK, D), jnp.float32),
            pltpu.VMEM_SHARED((N + NSINK, D), jnp.float32),
            pltpu.SemaphoreType.DMA,
            pltpu.SemaphoreType.DMA,
        ],
    )
    return kern(table, src, dst)


ROWS_BLK = 1000
GRID = N // ROWS_BLK


def _mlp1_body(x_ref, pa_ref, w1_ref, b1_ref, w2_ref, b2_ref, out_ref):
    h = x_ref[...] + pa_ref[0] + pa_ref[1]
    t = jnp.maximum(
        jnp.dot(h, w1_ref[...], preferred_element_type=jnp.float32)
        + b1_ref[...], 0.0)
    o = (jnp.dot(t, w2_ref[...], preferred_element_type=jnp.float32)
         + b2_ref[...])
    out_ref[...] = jnp.maximum(o, 0.0)


def _mlp1(x, pa, w1, b1, w2, b2):
    return pl.pallas_call(
        _mlp1_body,
        grid=(GRID,),
        in_specs=[
            pl.BlockSpec((ROWS_BLK, D), lambda i: (i, 0)),
            pl.BlockSpec((NC, ROWS_BLK, D), lambda i: (0, i, 0)),
            pl.BlockSpec((D, D), lambda i: (0, 0)),
            pl.BlockSpec((1, D), lambda i: (0, 0)),
            pl.BlockSpec((D, D), lambda i: (0, 0)),
            pl.BlockSpec((1, D), lambda i: (0, 0)),
        ],
        out_specs=pl.BlockSpec((ROWS_BLK, D), lambda i: (i, 0)),
        out_shape=jax.ShapeDtypeStruct((N, D), jnp.float32),
    )(x, pa, w1, b1, w2, b2)


def _mlp2_body(h_ref, pa_ref, w1_ref, b1_ref, w2_ref, b2_ref,
               e_ref, g_ref, out_ref):
    i = pl.program_id(0)
    hin = h_ref[...] + pa_ref[0] + pa_ref[1]
    t = jnp.maximum(
        jnp.dot(hin, w1_ref[...], preferred_element_type=jnp.float32)
        + b1_ref[...], 0.0)
    h2 = (jnp.dot(t, w2_ref[...], preferred_element_type=jnp.float32)
          + b2_ref[...])
    contrib = lax.dot_general(e_ref[...], h2, (((0,), (0,)), ((), ())),
                              preferred_element_type=jnp.float32)

    @pl.when(i == 0)
    def _():
        out_ref[...] = jnp.zeros_like(out_ref)

    out_ref[...] += contrib

    @pl.when(i == pl.num_programs(0) - 1)
    def _():
        den = jnp.sum(g_ref[...], axis=1, keepdims=True) + 1e-4
        out_ref[...] = out_ref[...] / den


def _mlp2(h, pa, w1, b1, w2, b2, e_rep, ghg):
    return pl.pallas_call(
        _mlp2_body,
        grid=(GRID,),
        in_specs=[
            pl.BlockSpec((ROWS_BLK, D), lambda i: (i, 0)),
            pl.BlockSpec((NC, ROWS_BLK, D), lambda i: (0, i, 0)),
            pl.BlockSpec((D, D), lambda i: (0, 0)),
            pl.BlockSpec((1, D), lambda i: (0, 0)),
            pl.BlockSpec((D, D), lambda i: (0, 0)),
            pl.BlockSpec((1, D), lambda i: (0, 0)),
            pl.BlockSpec((ROWS_BLK, G), lambda i: (i, 0)),
            pl.BlockSpec((G, N // GSZ), lambda i: (0, 0)),
        ],
        out_specs=pl.BlockSpec((G, D), lambda i: (0, 0)),
        out_shape=jax.ShapeDtypeStruct((G, D), jnp.float32),
    )(h, pa, w1, b1, w2, b2, e_rep, ghg)


def kernel(x, edge_index, graph_has_graphlet,
           W1a, b1a, W2a, b2a, W1b, b1b, W2b, b2b):
    pad = EP - E
    src = jnp.concatenate(
        [edge_index[0], jnp.zeros((pad,), jnp.int32)]).reshape(NCHUNKS_P,
                                                               CHUNK)
    dst = jnp.concatenate(
        [edge_index[1], jnp.full((pad,), N, jnp.int32)]).reshape(NCHUNKS_P,
                                                                 CHUNK)
    b1a2 = b1a.reshape(1, D)
    b2a2 = b2a.reshape(1, D)
    b1b2 = b1b.reshape(1, D)
    b2b2 = b2b.reshape(1, D)
    ghg_rep_t = jnp.repeat(graph_has_graphlet.T, GSZ, axis=0)

    pa1 = _sc_pass(x, src, dst)
    h1r = _mlp1(x, pa1, W1a, b1a2, W2a, b2a2)
    pa2 = _sc_pass(h1r, src, dst)
    out = _mlp2(h1r, pa2, W1b, b1b2, W2b, b2b2, ghg_rep_t, graph_has_graphlet)
    return out


# speedup vs baseline: 1.1003x; 1.1003x over previous
import functools

import jax
import jax.numpy as jnp
from jax import lax
from jax.experimental import pallas as pl
from jax.experimental.pallas import tpu as pltpu
from jax.experimental.pallas import tpu_sc as plsc

N = 10000
E = 320000
D = 128
G = 64
GSZ = 5

NC = 2
NS = 16
NW = NC * NS
CHUNK = 128
CPW = 80
NCHUNKS_P = NW * CPW
EP = NCHUNKS_P * CHUNK
NSINK = 8
RB = 200
NB = N // RB


def _sc_agg_body(table_hbm, src_hbm, dst_hbm, out_hbm,
                 src0_v, src1_v, dst0_v, dst1_v, rows0_v, rows1_v, acc_sh,
                 gsem0, gsem1):
    c = lax.axis_index("c")
    s = lax.axis_index("s")
    w = s * NC + c

    zero16 = jnp.zeros((16,), jnp.float32)

    def _zero_body(i, carry):
        for j in range(D // 16):
            rows0_v[i, pl.ds(j * 16, 16)] = zero16
        return carry

    lax.fori_loop(0, CHUNK, _zero_body, 0)

    nzb = N // CHUNK

    def _zinit(k, carry):
        bid = s + NS * k

        @pl.when(bid < nzb)
        def _():
            off = pl.multiple_of(bid * CHUNK, 8)
            pltpu.sync_copy(rows0_v, acc_sh.at[pl.ds(off, CHUNK)])

        return carry

    lax.fori_loop(0, (nzb + NS - 1) // NS, _zinit, 0)

    @pl.when(s == 0)
    def _():
        pltpu.sync_copy(rows0_v.at[pl.ds(0, 16)],
                        acc_sh.at[pl.ds(nzb * CHUNK, 16)])

    plsc.subcore_barrier()

    base0 = w * CPW * CHUNK

    def _ldidx(k, sbuf, dbuf):
        off = pl.multiple_of(base0 + k * CHUNK, 8)
        pltpu.sync_copy(src_hbm.at[pl.ds(off, CHUNK)], sbuf)
        pltpu.sync_copy(dst_hbm.at[pl.ds(off, CHUNK)], dbuf)

    def _gather(sbuf, buf, sem):
        return pltpu.async_copy(table_hbm.at[sbuf], buf, sem)

    def _gwait(sbuf, buf, sem):
        pltpu.make_async_copy(table_hbm.at[sbuf], buf, sem).wait()

    def _scat(dbuf, buf):
        pltpu.sync_copy(buf, acc_sh.at[dbuf], add=True)

    _ldidx(0, src0_v, dst0_v)
    _gather(src0_v, rows0_v, gsem0)

    def _edge_body(i, carry):
        a = 2 * i
        b = a + 1
        _ldidx(b, src1_v, dst1_v)
        _gather(src1_v, rows1_v, gsem1)
        _gwait(src0_v, rows0_v, gsem0)
        _scat(dst0_v, rows0_v)

        @pl.when(b + 1 < CPW)
        def _():
            _ldidx(b + 1, src0_v, dst0_v)
            _gather(src0_v, rows0_v, gsem0)

        _gwait(src1_v, rows1_v, gsem1)
        _scat(dst1_v, rows1_v)
        return carry

    lax.fori_loop(0, CPW // 2, _edge_body, 0)
    plsc.subcore_barrier()

    def _dump(k, carry):
        bid = s + NS * k

        @pl.when(bid < NB)
        def _():
            off = pl.multiple_of(bid * RB, 8)
            pltpu.sync_copy(acc_sh.at[pl.ds(off, RB)],
                            out_hbm.at[c, pl.ds(off, RB)])

        return carry

    lax.fori_loop(0, (NB + NS - 1) // NS, _dump, 0)


def _sc_pass(table, src, dst):
    mesh = plsc.VectorSubcoreMesh(core_axis_name="c", subcore_axis_name="s")
    kern = pl.kernel(
        _sc_agg_body,
        mesh=mesh,
        out_type=jax.ShapeDtypeStruct((NC, N, D), jnp.float32),
        scratch_types=[
            pltpu.VMEM((CHUNK,), jnp.int32),
            pltpu.VMEM((CHUNK,), jnp.int32),
            pltpu.VMEM((CHUNK,), jnp.int32),
            pltpu.VMEM((CHUNK,), jnp.int32),
            pltpu.VMEM((CHUNK, D), jnp.float32),
            pltpu.VMEM((CHUNK, D), jnp.float32),
            pltpu.VMEM_SHARED((N + NSINK, D), jnp.float32),
            pltpu.SemaphoreType.DMA,
            pltpu.SemaphoreType.DMA,
        ],
    )
    return kern(table, src, dst)


ROWS_BLK = 1000
GRID = N // ROWS_BLK


def _mlp1_body(x_ref, pa_ref, w1_ref, b1_ref, w2_ref, b2_ref, out_ref):
    h = x_ref[...] + pa_ref[0] + pa_ref[1]
    t = jnp.maximum(
        jnp.dot(h, w1_ref[...], preferred_element_type=jnp.float32)
        + b1_ref[...], 0.0)
    o = (jnp.dot(t, w2_ref[...], preferred_element_type=jnp.float32)
         + b2_ref[...])
    out_ref[...] = jnp.maximum(o, 0.0)


def _mlp1(x, pa, w1, b1, w2, b2):
    return pl.pallas_call(
        _mlp1_body,
        grid=(GRID,),
        in_specs=[
            pl.BlockSpec((ROWS_BLK, D), lambda i: (i, 0)),
            pl.BlockSpec((NC, ROWS_BLK, D), lambda i: (0, i, 0)),
            pl.BlockSpec((D, D), lambda i: (0, 0)),
            pl.BlockSpec((1, D), lambda i: (0, 0)),
            pl.BlockSpec((D, D), lambda i: (0, 0)),
            pl.BlockSpec((1, D), lambda i: (0, 0)),
        ],
        out_specs=pl.BlockSpec((ROWS_BLK, D), lambda i: (i, 0)),
        out_shape=jax.ShapeDtypeStruct((N, D), jnp.float32),
    )(x, pa, w1, b1, w2, b2)


def _mlp2_body(h_ref, pa_ref, w1_ref, b1_ref, w2_ref, b2_ref,
               e_ref, g_ref, out_ref):
    i = pl.program_id(0)
    hin = h_ref[...] + pa_ref[0] + pa_ref[1]
    t = jnp.maximum(
        jnp.dot(hin, w1_ref[...], preferred_element_type=jnp.float32)
        + b1_ref[...], 0.0)
    h2 = (jnp.dot(t, w2_ref[...], preferred_element_type=jnp.float32)
          + b2_ref[...])
    contrib = lax.dot_general(e_ref[...], h2, (((0,), (0,)), ((), ())),
                              preferred_element_type=jnp.float32)

    @pl.when(i == 0)
    def _():
        out_ref[...] = jnp.zeros_like(out_ref)

    out_ref[...] += contrib

    @pl.when(i == pl.num_programs(0) - 1)
    def _():
        den = jnp.sum(g_ref[...], axis=1, keepdims=True) + 1e-4
        out_ref[...] = out_ref[...] / den


def _mlp2(h, pa, w1, b1, w2, b2, e_rep, ghg):
    return pl.pallas_call(
        _mlp2_body,
        grid=(GRID,),
        in_specs=[
            pl.BlockSpec((ROWS_BLK, D), lambda i: (i, 0)),
            pl.BlockSpec((NC, ROWS_BLK, D), lambda i: (0, i, 0)),
            pl.BlockSpec((D, D), lambda i: (0, 0)),
            pl.BlockSpec((1, D), lambda i: (0, 0)),
            pl.BlockSpec((D, D), lambda i: (0, 0)),
            pl.BlockSpec((1, D), lambda i: (0, 0)),
            pl.BlockSpec((ROWS_BLK, G), lambda i: (i, 0)),
            pl.BlockSpec((G, N // GSZ), lambda i: (0, 0)),
        ],
        out_specs=pl.BlockSpec((G, D), lambda i: (0, 0)),
        out_shape=jax.ShapeDtypeStruct((G, D), jnp.float32),
    )(h, pa, w1, b1, w2, b2, e_rep, ghg)


def kernel(x, edge_index, graph_has_graphlet,
           W1a, b1a, W2a, b2a, W1b, b1b, W2b, b2b):
    pad = EP - E
    src = jnp.concatenate([edge_index[0], jnp.zeros((pad,), jnp.int32)])
    dst = jnp.concatenate([edge_index[1], jnp.full((pad,), N, jnp.int32)])
    b1a2 = b1a.reshape(1, D)
    b2a2 = b2a.reshape(1, D)
    b1b2 = b1b.reshape(1, D)
    b2b2 = b2b.reshape(1, D)
    ghg_rep_t = jnp.repeat(graph_has_graphlet.T, GSZ, axis=0)

    pa1 = _sc_pass(x, src, dst)
    h1r = _mlp1(x, pa1, W1a, b1a2, W2a, b2a2)
    pa2 = _sc_pass(h1r, src, dst)
    out = _mlp2(h1r, pa2, W1b, b1b2, W2b, b2b2, ghg_rep_t, graph_has_graphlet)
    return out


# speedup vs baseline: 3.2000x; 2.9084x over previous
import functools

import jax
import jax.numpy as jnp
from jax import lax
from jax.experimental import pallas as pl
from jax.experimental.pallas import tpu as pltpu
from jax.experimental.pallas import tpu_sc as plsc

N = 10000
E = 320000
D = 128
G = 64
GSZ = 5

NC = 2
NS = 16
NW = NC * NS
CHUNK = 128
NCHUNKS = E // CHUNK
RB = 200
NB = N // RB


def _sc_agg_body(table_hbm, src_hbm, dst_hbm, out_hbm,
                 src0_v, src1_v, dst0_v, dst1_v, rows0_v, rows1_v, acc_sh,
                 gsem0, gsem1):
    c = lax.axis_index("c")
    s = lax.axis_index("s")
    w = s * NC + c

    zero16 = jnp.zeros((16,), jnp.float32)

    def _zero_body(i, carry):
        for j in range(D // 16):
            rows0_v[i, pl.ds(j * 16, 16)] = zero16
        return carry

    lax.fori_loop(0, CHUNK, _zero_body, 0)

    nzb = N // CHUNK

    def _zinit(k, carry):
        bid = s + NS * k

        @pl.when(bid < nzb)
        def _():
            off = pl.multiple_of(bid * CHUNK, 8)
            pltpu.sync_copy(rows0_v, acc_sh.at[pl.ds(off, CHUNK)])

        return carry

    lax.fori_loop(0, (nzb + NS - 1) // NS, _zinit, 0)

    @pl.when(s == 0)
    def _():
        pltpu.sync_copy(rows0_v.at[pl.ds(0, 16)],
                        acc_sh.at[pl.ds(nzb * CHUNK, 16)])

    plsc.subcore_barrier()

    def _ldidx(cid, sbuf, dbuf):
        off = pl.multiple_of(cid * CHUNK, 8)
        pltpu.sync_copy(src_hbm.at[pl.ds(off, CHUNK)], sbuf)
        pltpu.sync_copy(dst_hbm.at[pl.ds(off, CHUNK)], dbuf)

    def _gather(sbuf, buf, sem):
        return pltpu.async_copy(table_hbm.at[sbuf], buf, sem)

    def _gwait(sbuf, buf, sem):
        pltpu.make_async_copy(table_hbm.at[sbuf], buf, sem).wait()

    def _scat(dbuf, buf):
        pltpu.sync_copy(buf, acc_sh.at[dbuf], add=True)

    _ldidx(w, src0_v, dst0_v)
    _gather(src0_v, rows0_v, gsem0)

    def _edge_body(i, carry):
        ca = w + (2 * i) * NW
        cb = ca + NW

        @pl.when(cb < NCHUNKS)
        def _():
            _ldidx(cb, src1_v, dst1_v)
            _gather(src1_v, rows1_v, gsem1)

        @pl.when(ca < NCHUNKS)
        def _():
            _gwait(src0_v, rows0_v, gsem0)
            _scat(dst0_v, rows0_v)

        @pl.when(cb + NW < NCHUNKS)
        def _():
            _ldidx(cb + NW, src0_v, dst0_v)
            _gather(src0_v, rows0_v, gsem0)

        @pl.when(cb < NCHUNKS)
        def _():
            _gwait(src1_v, rows1_v, gsem1)
            _scat(dst1_v, rows1_v)

        return carry

    lax.fori_loop(0, (NCHUNKS // NW + 2) // 2, _edge_body, 0)
    plsc.subcore_barrier()

    def _dump(k, carry):
        bid = s + NS * k

        @pl.when(bid < NB)
        def _():
            off = pl.multiple_of(bid * RB, 8)
            pltpu.sync_copy(acc_sh.at[pl.ds(off, RB)],
                            out_hbm.at[c, pl.ds(off, RB)])

        return carry

    lax.fori_loop(0, (NB + NS - 1) // NS, _dump, 0)


def _sc_pass(table, src, dst):
    mesh = plsc.VectorSubcoreMesh(core_axis_name="c", subcore_axis_name="s")
    kern = pl.kernel(
        _sc_agg_body,
        mesh=mesh,
        out_type=jax.ShapeDtypeStruct((NC, N, D), jnp.float32),
        scratch_types=[
            pltpu.VMEM((CHUNK,), jnp.int32),
            pltpu.VMEM((CHUNK,), jnp.int32),
            pltpu.VMEM((CHUNK,), jnp.int32),
            pltpu.VMEM((CHUNK,), jnp.int32),
            pltpu.VMEM((CHUNK, D), jnp.float32),
            pltpu.VMEM((CHUNK, D), jnp.float32),
            pltpu.VMEM_SHARED((N, D), jnp.float32),
            pltpu.SemaphoreType.DMA,
            pltpu.SemaphoreType.DMA,
        ],
    )
    return kern(table, src, dst)


ROWS_BLK = 1000
GRID = N // ROWS_BLK


def _mlp1_body(x_ref, pa_ref, w1_ref, b1_ref, w2_ref, b2_ref, out_ref):
    h = x_ref[...] + pa_ref[0] + pa_ref[1]
    t = jnp.maximum(
        jnp.dot(h, w1_ref[...], preferred_element_type=jnp.float32)
        + b1_ref[...], 0.0)
    o = (jnp.dot(t, w2_ref[...], preferred_element_type=jnp.float32)
         + b2_ref[...])
    out_ref[...] = jnp.maximum(o, 0.0)


def _mlp1(x, pa, w1, b1, w2, b2):
    return pl.pallas_call(
        _mlp1_body,
        grid=(GRID,),
        in_specs=[
            pl.BlockSpec((ROWS_BLK, D), lambda i: (i, 0)),
            pl.BlockSpec((NC, ROWS_BLK, D), lambda i: (0, i, 0)),
            pl.BlockSpec((D, D), lambda i: (0, 0)),
            pl.BlockSpec((1, D), lambda i: (0, 0)),
            pl.BlockSpec((D, D), lambda i: (0, 0)),
            pl.BlockSpec((1, D), lambda i: (0, 0)),
        ],
        out_specs=pl.BlockSpec((ROWS_BLK, D), lambda i: (i, 0)),
        out_shape=jax.ShapeDtypeStruct((N, D), jnp.float32),
    )(x, pa, w1, b1, w2, b2)


def _mlp2_body(h_ref, pa_ref, w1_ref, b1_ref, w2_ref, b2_ref,
               e_ref, g_ref, out_ref):
    i = pl.program_id(0)
    hin = h_ref[...] + pa_ref[0] + pa_ref[1]
    t = jnp.maximum(
        jnp.dot(hin, w1_ref[...], preferred_element_type=jnp.float32)
        + b1_ref[...], 0.0)
    h2 = (jnp.dot(t, w2_ref[...], preferred_element_type=jnp.float32)
          + b2_ref[...])
    contrib = lax.dot_general(e_ref[...], h2, (((0,), (0,)), ((), ())),
                              preferred_element_type=jnp.float32)

    @pl.when(i == 0)
    def _():
        out_ref[...] = jnp.zeros_like(out_ref)

    out_ref[...] += contrib

    @pl.when(i == pl.num_programs(0) - 1)
    def _():
        den = jnp.sum(g_ref[...], axis=1, keepdims=True) + 1e-4
        out_ref[...] = out_ref[...] / den


def _mlp2(h, pa, w1, b1, w2, b2, e_rep, ghg):
    return pl.pallas_call(
        _mlp2_body,
        grid=(GRID,),
        in_specs=[
            pl.BlockSpec((ROWS_BLK, D), lambda i: (i, 0)),
            pl.BlockSpec((NC, ROWS_BLK, D), lambda i: (0, i, 0)),
            pl.BlockSpec((D, D), lambda i: (0, 0)),
            pl.BlockSpec((1, D), lambda i: (0, 0)),
            pl.BlockSpec((D, D), lambda i: (0, 0)),
            pl.BlockSpec((1, D), lambda i: (0, 0)),
            pl.BlockSpec((ROWS_BLK, G), lambda i: (i, 0)),
            pl.BlockSpec((G, N // GSZ), lambda i: (0, 0)),
        ],
        out_specs=pl.BlockSpec((G, D), lambda i: (0, 0)),
        out_shape=jax.ShapeDtypeStruct((G, D), jnp.float32),
    )(h, pa, w1, b1, w2, b2, e_rep, ghg)


def kernel(x, edge_index, graph_has_graphlet,
           W1a, b1a, W2a, b2a, W1b, b1b, W2b, b2b):
    src = edge_index[0]
    dst = edge_index[1]
    b1a2 = b1a.reshape(1, D)
    b2a2 = b2a.reshape(1, D)
    b1b2 = b1b.reshape(1, D)
    b2b2 = b2b.reshape(1, D)
    ghg_rep_t = jnp.repeat(graph_has_graphlet.T, GSZ, axis=0)

    pa1 = _sc_pass(x, src, dst)
    h1r = _mlp1(x, pa1, W1a, b1a2, W2a, b2a2)
    pa2 = _sc_pass(h1r, src, dst)
    out = _mlp2(h1r, pa2, W1b, b1b2, W2b, b2b2, ghg_rep_t, graph_has_graphlet)
    return out
